# Initial kernel scaffold; baseline (speedup 1.0000x reference)
import jax
import jax.numpy as jnp
from jax import lax
from jax.experimental import pallas as pl
from jax.experimental.pallas import tpu as pltpu

N_KEYS = 100000
BATCH = 1024
DIM = 64
KBLK = 2000
NBLK = N_KEYS // KBLK


def _attn_body(q_ref, vt_ref, g_ref, b_ref, k_ref, v_ref,
               ret_ref, sur_ref, qn_ref, dec_ref,
               m_ref, l_ref, acc_ref):
    i = pl.program_id(0)

    @pl.when(i == 0)
    def _init():
        q = q_ref[...]
        mu = jnp.mean(q, axis=1, keepdims=True)
        var = jnp.mean((q - mu) ** 2, axis=1, keepdims=True)
        qn = (q - mu) * lax.rsqrt(var + 1e-5) * g_ref[...] + b_ref[...]
        qn_ref[...] = qn
        m_ref[...] = jnp.full((BATCH, 1), -1e30, jnp.float32)
        l_ref[...] = jnp.zeros((BATCH, 1), jnp.float32)
        acc_ref[...] = jnp.zeros((BATCH, DIM), jnp.float32)

    qn = qn_ref[...]
    qsq = jnp.sum(qn * qn, axis=1, keepdims=True)
    k = k_ref[...]
    kk = jnp.sum(k * k, axis=1)[None, :]
    qk = lax.dot_general(qn, k, (((1,), (1,)), ((), ())),
                         preferred_element_type=jnp.float32,
                         precision=lax.Precision.HIGHEST)
    s = -jnp.maximum(qsq + kk - 2.0 * qk, 0.0)
    m_prev = m_ref[...]
    m_new = jnp.maximum(m_prev, jnp.max(s, axis=1, keepdims=True))
    alpha = jnp.exp(m_prev - m_new)
    p = jnp.exp(s - m_new)
    l_ref[...] = l_ref[...] * alpha + jnp.sum(p, axis=1, keepdims=True)
    acc_ref[...] = acc_ref[...] * alpha + lax.dot_general(
        p, v_ref[...], (((1,), (0,)), ((), ())),
        preferred_element_type=jnp.float32, precision=lax.Precision.HIGHEST)
    m_ref[...] = m_new

    @pl.when(i == NBLK - 1)
    def _fin():
        r = acc_ref[...] / l_ref[...]
        ret_ref[...] = r
        diff = r - vt_ref[...]
        sur = jnp.mean(diff * diff, axis=1, keepdims=True)
        sur_ref[...] = sur
        w = jax.nn.sigmoid(sur - jnp.mean(sur))
        dec_ref[...] = 0.99 * (1.0 - w)


def _attention(query, value_target, keys, values, gamma, beta):
    return pl.pallas_call(
        _attn_body,
        grid=(NBLK,),
        in_specs=[
            pl.BlockSpec((BATCH, DIM), lambda i: (0, 0)),
            pl.BlockSpec((BATCH, DIM), lambda i: (0, 0)),
            pl.BlockSpec((1, DIM), lambda i: (0, 0)),
            pl.BlockSpec((1, DIM), lambda i: (0, 0)),
            pl.BlockSpec((KBLK, DIM), lambda i: (i, 0)),
            pl.BlockSpec((KBLK, DIM), lambda i: (i, 0)),
        ],
        out_specs=[
            pl.BlockSpec((BATCH, DIM), lambda i: (0, 0)),
            pl.BlockSpec((BATCH, 1), lambda i: (0, 0)),
            pl.BlockSpec((BATCH, DIM), lambda i: (0, 0)),
            pl.BlockSpec((BATCH, 1), lambda i: (0, 0)),
        ],
        out_shape=[
            jax.ShapeDtypeStruct((BATCH, DIM), jnp.float32),
            jax.ShapeDtypeStruct((BATCH, 1), jnp.float32),
            jax.ShapeDtypeStruct((BATCH, DIM), jnp.float32),
            jax.ShapeDtypeStruct((BATCH, 1), jnp.float32),
        ],
        scratch_shapes=[
            pltpu.VMEM((BATCH, 1), jnp.float32),
            pltpu.VMEM((BATCH, 1), jnp.float32),
            pltpu.VMEM((BATCH, DIM), jnp.float32),
        ],
        compiler_params=pltpu.CompilerParams(
            dimension_semantics=("arbitrary",),
        ),
    )(query, value_target, gamma.reshape(1, DIM), beta.reshape(1, DIM),
      keys, values)


def kernel(query, value_target, keys, values, slot_age, kn_gamma, kn_beta):
    retrieved, sur, qn, dec = _attention(
        query, value_target, keys, values, kn_gamma, kn_beta)
    surprise = sur[:, 0]
    decay = dec[:, 0]
    _, oldest = lax.top_k(slot_age, BATCH)
    new_keys = keys.at[oldest].set(
        decay[:, None] * keys[oldest] + (1.0 - decay[:, None]) * qn)
    new_values = values.at[oldest].set(
        decay[:, None] * values[oldest] + (1.0 - decay[:, None]) * value_target)
    new_slot_age = slot_age.at[oldest].set(0.0) + 1.0
    return retrieved, surprise, new_keys, new_values, new_slot_age


# speedup vs baseline: 1.2096x; 1.2096x over previous
import jax
import jax.numpy as jnp
from jax import lax
from jax.experimental import pallas as pl
from jax.experimental.pallas import tpu as pltpu

N_KEYS = 100000
BATCH = 1024
DIM = 64
KBLK = 2000
NBLK = N_KEYS // KBLK


def _attn_body(q_ref, vt_ref, g_ref, b_ref, k_ref, v_ref,
               ret_ref, sur_ref, qn_ref, dec_ref,
               m_ref, l_ref, acc_ref):
    i = pl.program_id(0)

    @pl.when(i == 0)
    def _init():
        q = q_ref[...]
        mu = jnp.mean(q, axis=1, keepdims=True)
        var = jnp.mean((q - mu) ** 2, axis=1, keepdims=True)
        qn = (q - mu) * lax.rsqrt(var + 1e-5) * g_ref[...] + b_ref[...]
        qn_ref[...] = qn
        m_ref[...] = jnp.full((BATCH, 1), -1e30, jnp.float32)
        l_ref[...] = jnp.zeros((BATCH, 1), jnp.float32)
        acc_ref[...] = jnp.zeros((BATCH, DIM), jnp.float32)

    qn = qn_ref[...]
    qsq = jnp.sum(qn * qn, axis=1, keepdims=True)
    k = k_ref[...]
    kk = jnp.sum(k * k, axis=1)[None, :]
    qk = lax.dot_general(qn, k, (((1,), (1,)), ((), ())),
                         preferred_element_type=jnp.float32)
    s = -jnp.maximum(qsq + kk - 2.0 * qk, 0.0)
    m_prev = m_ref[...]
    m_new = jnp.maximum(m_prev, jnp.max(s, axis=1, keepdims=True))
    alpha = jnp.exp(m_prev - m_new)
    p = jnp.exp(s - m_new)
    l_ref[...] = l_ref[...] * alpha + jnp.sum(p, axis=1, keepdims=True)
    acc_ref[...] = acc_ref[...] * alpha + lax.dot_general(
        p, v_ref[...], (((1,), (0,)), ((), ())),
        preferred_element_type=jnp.float32)
    m_ref[...] = m_new

    @pl.when(i == NBLK - 1)
    def _fin():
        r = acc_ref[...] / l_ref[...]
        ret_ref[...] = r
        diff = r - vt_ref[...]
        sur = jnp.mean(diff * diff, axis=1, keepdims=True)
        sur_ref[...] = sur
        w = jax.nn.sigmoid(sur - jnp.mean(sur))
        dec_ref[...] = 0.99 * (1.0 - w)


def _attention(query, value_target, keys, values, gamma, beta):
    return pl.pallas_call(
        _attn_body,
        grid=(NBLK,),
        in_specs=[
            pl.BlockSpec((BATCH, DIM), lambda i: (0, 0)),
            pl.BlockSpec((BATCH, DIM), lambda i: (0, 0)),
            pl.BlockSpec((1, DIM), lambda i: (0, 0)),
            pl.BlockSpec((1, DIM), lambda i: (0, 0)),
            pl.BlockSpec((KBLK, DIM), lambda i: (i, 0)),
            pl.BlockSpec((KBLK, DIM), lambda i: (i, 0)),
        ],
        out_specs=[
            pl.BlockSpec((BATCH, DIM), lambda i: (0, 0)),
            pl.BlockSpec((BATCH, 1), lambda i: (0, 0)),
            pl.BlockSpec((BATCH, DIM), lambda i: (0, 0)),
            pl.BlockSpec((BATCH, 1), lambda i: (0, 0)),
        ],
        out_shape=[
            jax.ShapeDtypeStruct((BATCH, DIM), jnp.float32),
            jax.ShapeDtypeStruct((BATCH, 1), jnp.float32),
            jax.ShapeDtypeStruct((BATCH, DIM), jnp.float32),
            jax.ShapeDtypeStruct((BATCH, 1), jnp.float32),
        ],
        scratch_shapes=[
            pltpu.VMEM((BATCH, 1), jnp.float32),
            pltpu.VMEM((BATCH, 1), jnp.float32),
            pltpu.VMEM((BATCH, DIM), jnp.float32),
        ],
        compiler_params=pltpu.CompilerParams(
            dimension_semantics=("arbitrary",),
        ),
    )(query, value_target, gamma.reshape(1, DIM), beta.reshape(1, DIM),
      keys, values)


def kernel(query, value_target, keys, values, slot_age, kn_gamma, kn_beta):
    retrieved, sur, qn, dec = _attention(
        query, value_target, keys, values, kn_gamma, kn_beta)
    surprise = sur[:, 0]
    decay = dec[:, 0]
    _, oldest = lax.top_k(slot_age, BATCH)
    new_keys = keys.at[oldest].set(
        decay[:, None] * keys[oldest] + (1.0 - decay[:, None]) * qn)
    new_values = values.at[oldest].set(
        decay[:, None] * values[oldest] + (1.0 - decay[:, None]) * value_target)
    new_slot_age = slot_age.at[oldest].set(0.0) + 1.0
    return retrieved, surprise, new_keys, new_values, new_slot_age
